# baseline (speedup 1.0000x reference)
import functools

import jax
import jax.numpy as jnp
from jax import lax
from jax.experimental import pallas as pl
from jax.experimental.pallas import tpu as pltpu
from jax.experimental.pallas import tpu_sc as plsc

E = 8
TOPK = 2
BT = 256
_GELU_C = 0.7978845608028654


def _gelu_tanh(x):
    return 0.5 * x * (1.0 + jnp.tanh(_GELU_C * (x + 0.044715 * x * x * x)))


def _router_kernel(flat_ref, rw_ref, idx_ref, prob_ref):
    logits = lax.dot_general(
        rw_ref[...], flat_ref[...], (((1,), (1,)), ((), ())),
        preferred_element_type=jnp.float32)
    e, t = logits.shape
    ids = lax.broadcasted_iota(jnp.int32, (e, t), 0)
    m1 = jnp.max(logits, axis=0, keepdims=True)
    a1 = jnp.min(jnp.where(logits == m1, ids, e), axis=0, keepdims=True)
    l2 = jnp.where(ids == a1, -jnp.inf, logits)
    m2 = jnp.max(l2, axis=0, keepdims=True)
    a2 = jnp.min(jnp.where(l2 == m2, ids, e), axis=0, keepdims=True)
    e2 = jnp.exp(m2 - m1)
    p1 = 1.0 / (1.0 + e2)
    p2 = e2 * p1
    idx_ref[...] = jnp.concatenate([a1, a2], axis=0)
    prob_ref[...] = jnp.concatenate([p1, p2], axis=0)


def _dispatch_body(a_hbm, pr_hbm, tok_hbm, psort_hbm, pos_hbm, off_hbm,
                   a_v, pr_v, tok_v, psort_v, pos_v, run_v, off_v):
    wid = lax.axis_index("s") * 2 + lax.axis_index("c")
    na = a_v.shape[0]
    nb = tok_v.shape[0]

    @pl.when(wid == 0)
    def _():
        pltpu.sync_copy(a_hbm, a_v)
        pltpu.sync_copy(pr_hbm, pr_v)
        zi = jnp.zeros((16,), jnp.int32)
        run_v[...] = zi

        def zero_body(j, c):
            tok_v[pl.ds(j * 16, 16)] = zi
            psort_v[pl.ds(j * 16, 16)] = jnp.zeros((16,), jnp.float32)
            return c
        lax.fori_loop(0, nb // 16, zero_body, 0)

        ones = jnp.ones((16,), jnp.int32)

        def count_body(j, c):
            a = a_v[pl.ds(j * 16, 16)]
            r, last = plsc.scan_count(a)
            base = plsc.load_gather(run_v, [a])
            plsc.store_scatter(run_v, [a], base + r, mask=last)
            return c
        lax.fori_loop(0, na // 16, count_body, 0)

        cnt = run_v[...]
        pad = ((cnt + (BT - 1)) >> 8) << 8
        off_incl = plsc.cumsum(pad)
        off_v[...] = off_incl
        run_v[...] = off_incl - pad

        def pos_body(j, c):
            a = a_v[pl.ds(j * 16, 16)]
            r, last = plsc.scan_count(a)
            base = plsc.load_gather(run_v, [a])
            pos = base + r - 1
            plsc.store_scatter(run_v, [a], pos + 1, mask=last)
            pos_v[pl.ds(j * 16, 16)] = pos
            toks = (j * 16 + lax.broadcasted_iota(jnp.int32, (16,), 0)) >> 1
            plsc.store_scatter(tok_v, [pos], toks)
            plsc.store_scatter(psort_v, [pos], pr_v[pl.ds(j * 16, 16)])
            return c
        lax.fori_loop(0, na // 16, pos_body, 0)

        pltpu.sync_copy(tok_v, tok_hbm)
        pltpu.sync_copy(psort_v, psort_hbm)
        pltpu.sync_copy(pos_v, pos_hbm)
        pltpu.sync_copy(off_v, off_hbm)


def _gather_body(flat_hbm, tok_hbm, xs_hbm, idx_v, buf_a, buf_b,
                 gsem, ssem_a, ssem_b):
    wid = lax.axis_index("s") * 2 + lax.axis_index("c")
    rows = idx_v.shape[0]
    chunk = buf_a.shape[0]
    nch = rows // chunk
    base = wid * rows
    pltpu.sync_copy(tok_hbm.at[pl.ds(base, rows)], idx_v)
    bufs = (buf_a, buf_b)
    ssems = (ssem_a, ssem_b)
    handles = [None, None]
    for c in range(nch):
        buf = bufs[c % 2]
        if handles[c % 2] is not None:
            handles[c % 2].wait()
        pltpu.async_copy(
            flat_hbm.at[idx_v.at[pl.ds(c * chunk, chunk)]], buf, gsem).wait()
        handles[c % 2] = pltpu.async_copy(
            buf, xs_hbm.at[pl.ds(base + c * chunk, chunk)], ssems[c % 2])
    for h in handles:
        if h is not None:
            h.wait()


def _ffn_kernel(eid_ref, xs_ref, w1_ref, w2_ref, prob_ref, ys_ref,
                w1b_ref, w2b_ref):
    i = pl.program_id(0)
    e = eid_ref[i]
    prev = eid_ref[jnp.maximum(i - 1, 0)]

    @pl.when((i == 0) | (e != prev))
    def _():
        w1b_ref[...] = w1_ref[0].astype(jnp.bfloat16)
        w2b_ref[...] = w2_ref[0].astype(jnp.bfloat16)

    x = xs_ref[...].astype(jnp.bfloat16)
    h = jnp.dot(x, w1b_ref[...], preferred_element_type=jnp.float32)
    h = _gelu_tanh(h)
    y = jnp.dot(h.astype(jnp.bfloat16), w2b_ref[...],
                preferred_element_type=jnp.float32)
    ys_ref[...] = y * prob_ref[...]


def _combine_body(ys_hbm, p1_hbm, p2_hbm, out_hbm, i1_v, i2_v, buf_a, buf_b,
                  sem_a, sem_b, osem):
    wid = lax.axis_index("s") * 2 + lax.axis_index("c")
    ntok = i1_v.shape[0]
    chunk = buf_a.shape[0]
    nch = ntok // chunk
    h = buf_a.shape[1]
    base = wid * ntok
    pltpu.sync_copy(p1_hbm.at[pl.ds(base, ntok)], i1_v)
    pltpu.sync_copy(p2_hbm.at[pl.ds(base, ntok)], i2_v)
    for c in range(nch):
        ca = pltpu.async_copy(ys_hbm.at[i1_v.at[pl.ds(c * chunk, chunk)]],
                              buf_a, sem_a)
        cb = pltpu.async_copy(ys_hbm.at[i2_v.at[pl.ds(c * chunk, chunk)]],
                              buf_b, sem_b)
        ca.wait()
        cb.wait()

        def add_body(j, carry):
            r = j // (h // 16)
            o = (j % (h // 16)) * 16
            buf_a[r, pl.ds(o, 16)] = buf_a[r, pl.ds(o, 16)] + buf_b[r, pl.ds(o, 16)]
            return carry
        lax.fori_loop(0, chunk * (h // 16), add_body, 0)
        pltpu.async_copy(buf_a, out_hbm.at[pl.ds(base + c * chunk, chunk)],
                         osem).wait()


def kernel(hidden_states, router_weight, w1, w2):
    b, s, h = hidden_states.shape
    t = b * s
    dff = w1.shape[2]
    na = t * TOPK
    nblk = na // BT + E
    nrow = nblk * BT
    flat = hidden_states.reshape(t, h)

    idx2, prob2 = pl.pallas_call(
        _router_kernel,
        out_shape=(jax.ShapeDtypeStruct((2, t), jnp.int32),
                   jax.ShapeDtypeStruct((2, t), jnp.float32)),
    )(flat, router_weight)
    a_flat = idx2.T.reshape(na)
    pr_flat = prob2.T.reshape(na)

    mesh = plsc.VectorSubcoreMesh(core_axis_name="c", subcore_axis_name="s",
                                  num_cores=2)
    nw = 32

    tok_sorted, prob_sorted, pos, off_incl = pl.kernel(
        _dispatch_body,
        out_type=(jax.ShapeDtypeStruct((nrow,), jnp.int32),
                  jax.ShapeDtypeStruct((nrow,), jnp.float32),
                  jax.ShapeDtypeStruct((na,), jnp.int32),
                  jax.ShapeDtypeStruct((16,), jnp.int32)),
        mesh=mesh,
        compiler_params=pltpu.CompilerParams(needs_layout_passes=False),
        scratch_types=[
            pltpu.VMEM((na,), jnp.int32),
            pltpu.VMEM((na,), jnp.float32),
            pltpu.VMEM((nrow,), jnp.int32),
            pltpu.VMEM((nrow,), jnp.float32),
            pltpu.VMEM((na,), jnp.int32),
            pltpu.VMEM((16,), jnp.int32),
            pltpu.VMEM((16,), jnp.int32),
        ],
    )(a_flat, pr_flat)

    xs = pl.kernel(
        _gather_body,
        out_type=jax.ShapeDtypeStruct((nrow, h), jnp.float32),
        mesh=mesh,
        compiler_params=pltpu.CompilerParams(needs_layout_passes=False),
        scratch_types=[
            pltpu.VMEM((nrow // nw,), jnp.int32),
            pltpu.VMEM((48, h), jnp.float32),
            pltpu.VMEM((48, h), jnp.float32),
            pltpu.SemaphoreType.DMA,
            pltpu.SemaphoreType.DMA,
            pltpu.SemaphoreType.DMA,
        ],
    )(flat, tok_sorted)

    blk = jnp.arange(nblk, dtype=jnp.int32) * BT
    eid = jnp.sum((blk[:, None] >= off_incl[None, :E]).astype(jnp.int32),
                  axis=1)
    eid = jnp.minimum(eid, E - 1)

    ys = pl.pallas_call(
        _ffn_kernel,
        grid_spec=pltpu.PrefetchScalarGridSpec(
            num_scalar_prefetch=1,
            grid=(nblk,),
            in_specs=[
                pl.BlockSpec((BT, h), lambda i, eid_ref: (i, 0)),
                pl.BlockSpec((1, h, dff), lambda i, eid_ref: (eid_ref[i], 0, 0)),
                pl.BlockSpec((1, dff, h), lambda i, eid_ref: (eid_ref[i], 0, 0)),
                pl.BlockSpec((BT, 1), lambda i, eid_ref: (i, 0)),
            ],
            out_specs=pl.BlockSpec((BT, h), lambda i, eid_ref: (i, 0)),
            scratch_shapes=[
                pltpu.VMEM((h, dff), jnp.bfloat16),
                pltpu.VMEM((dff, h), jnp.bfloat16),
            ],
        ),
        out_shape=jax.ShapeDtypeStruct((nrow, h), jnp.float32),
    )(eid, xs, w1, w2, prob_sorted.reshape(nrow, 1))

    pos2d = pos.reshape(t, TOPK)
    pos_a = pos2d[:, 0]
    pos_b = pos2d[:, 1]

    out = pl.kernel(
        _combine_body,
        out_type=jax.ShapeDtypeStruct((t, h), jnp.float32),
        mesh=mesh,
        compiler_params=pltpu.CompilerParams(needs_layout_passes=False),
        scratch_types=[
            pltpu.VMEM((t // nw,), jnp.int32),
            pltpu.VMEM((t // nw,), jnp.int32),
            pltpu.VMEM((32, h), jnp.float32),
            pltpu.VMEM((32, h), jnp.float32),
            pltpu.SemaphoreType.DMA,
            pltpu.SemaphoreType.DMA,
            pltpu.SemaphoreType.DMA,
        ],
    )(ys, pos_a, pos_b)
    return out.reshape(b, s, h)


# speedup vs baseline: 1.0536x; 1.0536x over previous
import functools

import jax
import jax.numpy as jnp
from jax import lax
from jax.experimental import pallas as pl
from jax.experimental.pallas import tpu as pltpu
from jax.experimental.pallas import tpu_sc as plsc

E = 8
TOPK = 2
BT = 256
_GELU_C = 0.7978845608028654


def _gelu_tanh(x):
    return 0.5 * x * (1.0 + jnp.tanh(_GELU_C * (x + 0.044715 * x * x * x)))


def _router_kernel(flat_ref, rw_ref, idx_ref, prob_ref):
    logits = lax.dot_general(
        rw_ref[...], flat_ref[...], (((1,), (1,)), ((), ())),
        preferred_element_type=jnp.float32)
    e, t = logits.shape
    ids = lax.broadcasted_iota(jnp.int32, (e, t), 0)
    m1 = jnp.max(logits, axis=0, keepdims=True)
    a1 = jnp.min(jnp.where(logits == m1, ids, e), axis=0, keepdims=True)
    l2 = jnp.where(ids == a1, -jnp.inf, logits)
    m2 = jnp.max(l2, axis=0, keepdims=True)
    a2 = jnp.min(jnp.where(l2 == m2, ids, e), axis=0, keepdims=True)
    e2 = jnp.exp(m2 - m1)
    p1 = 1.0 / (1.0 + e2)
    p2 = e2 * p1
    idx_ref[...] = jnp.concatenate([a1, a2], axis=0)
    prob_ref[...] = jnp.concatenate([p1, p2], axis=0)


def _dispatch_body(a_hbm, pr_hbm, tok_hbm, psort_hbm, pos_hbm, off_hbm,
                   a_v, pr_v, tok_v, psort_v, pos_v, run_v, off_v):
    wid = lax.axis_index("s") * 2 + lax.axis_index("c")
    na = a_v.shape[0]
    nb = tok_v.shape[0]

    @pl.when(wid == 0)
    def _():
        pltpu.sync_copy(a_hbm, a_v)
        pltpu.sync_copy(pr_hbm, pr_v)
        zi = jnp.zeros((16,), jnp.int32)
        run_v[...] = zi

        @plsc.parallel_loop(0, nb, step=16, unroll=8)
        def _(j):
            tok_v[pl.ds(j, 16)] = jnp.zeros((16,), jnp.int32)
            psort_v[pl.ds(j, 16)] = jnp.zeros((16,), jnp.float32)

        def count_body(j, c):
            a = a_v[pl.ds(j * 16, 16)]
            r, last = plsc.scan_count(a)
            base = plsc.load_gather(run_v, [a])
            plsc.store_scatter(run_v, [a], base + r, mask=last)
            return c
        lax.fori_loop(0, na // 16, count_body, 0)

        cnt = run_v[...]
        pad = ((cnt + (BT - 1)) >> 8) << 8
        off_incl = plsc.cumsum(pad)
        off_v[...] = off_incl
        run_v[...] = off_incl - pad

        def pos_body(j, c):
            a = a_v[pl.ds(j * 16, 16)]
            r, last = plsc.scan_count(a)
            base = plsc.load_gather(run_v, [a])
            pos = base + r - 1
            plsc.store_scatter(run_v, [a], pos + 1, mask=last)
            pos_v[pl.ds(j * 16, 16)] = pos
            toks = (j * 16 + lax.broadcasted_iota(jnp.int32, (16,), 0)) >> 1
            plsc.store_scatter(tok_v, [pos], toks)
            plsc.store_scatter(psort_v, [pos], pr_v[pl.ds(j * 16, 16)])
            return c
        lax.fori_loop(0, na // 16, pos_body, 0)

        pltpu.sync_copy(tok_v, tok_hbm)
        pltpu.sync_copy(psort_v, psort_hbm)
        pltpu.sync_copy(pos_v, pos_hbm)
        pltpu.sync_copy(off_v, off_hbm)


def _gather_body(flat_hbm, tok_hbm, xs_hbm, idx_a, idx_b, idx_c, idx_d,
                 buf_a, buf_b, gsem, ssem_a, ssem_b):
    wid = lax.axis_index("s") * 2 + lax.axis_index("c")
    chunk = buf_a.shape[0]
    idxs = (idx_a, idx_b, idx_c, idx_d)
    nch = len(idxs)
    rows = chunk * nch
    base = wid * rows
    for c in range(nch):
        pltpu.sync_copy(tok_hbm.at[pl.ds(base + c * chunk, chunk)], idxs[c])
    bufs = (buf_a, buf_b)
    ssems = (ssem_a, ssem_b)
    handles = [None, None]
    for c in range(nch):
        buf = bufs[c % 2]
        if handles[c % 2] is not None:
            handles[c % 2].wait()
        pltpu.async_copy(flat_hbm.at[idxs[c]], buf, gsem).wait()
        handles[c % 2] = pltpu.async_copy(
            buf, xs_hbm.at[pl.ds(base + c * chunk, chunk)], ssems[c % 2])
    for h in handles:
        if h is not None:
            h.wait()


def _ffn_kernel(eid_ref, xs_ref, w1_ref, w2_ref, prob_ref, ys_ref,
                w1b_ref, w2b_ref):
    i = pl.program_id(0)
    e = eid_ref[i]
    prev = eid_ref[jnp.maximum(i - 1, 0)]

    @pl.when((i == 0) | (e != prev))
    def _():
        w1b_ref[...] = w1_ref[0].astype(jnp.bfloat16)
        w2b_ref[...] = w2_ref[0].astype(jnp.bfloat16)

    x = xs_ref[...].astype(jnp.bfloat16)
    h = jnp.dot(x, w1b_ref[...], preferred_element_type=jnp.float32)
    h = _gelu_tanh(h)
    y = jnp.dot(h.astype(jnp.bfloat16), w2b_ref[...],
                preferred_element_type=jnp.float32)
    ys_ref[...] = y * prob_ref[...]


def _combine_body(ys_hbm, p1_hbm, p2_hbm, out_hbm, i1_a, i1_b, i2_a, i2_b,
                  buf_a, buf_b, sem_a, sem_b, osem):
    wid = lax.axis_index("s") * 2 + lax.axis_index("c")
    chunk = buf_a.shape[0]
    i1s = (i1_a, i1_b)
    i2s = (i2_a, i2_b)
    nch = len(i1s)
    ntok = chunk * nch
    h = buf_a.shape[1]
    base = wid * ntok
    for c in range(nch):
        pltpu.sync_copy(p1_hbm.at[pl.ds(base + c * chunk, chunk)], i1s[c])
        pltpu.sync_copy(p2_hbm.at[pl.ds(base + c * chunk, chunk)], i2s[c])
    for c in range(nch):
        ca = pltpu.async_copy(ys_hbm.at[i1s[c]], buf_a, sem_a)
        cb = pltpu.async_copy(ys_hbm.at[i2s[c]], buf_b, sem_b)
        ca.wait()
        cb.wait()

        @plsc.parallel_loop(0, chunk * h, step=16, unroll=8)
        def _(j):
            r = j // h
            o = j % h
            buf_a[r, pl.ds(o, 16)] = buf_a[r, pl.ds(o, 16)] + buf_b[r, pl.ds(o, 16)]
        pltpu.async_copy(buf_a, out_hbm.at[pl.ds(base + c * chunk, chunk)],
                         osem).wait()


def kernel(hidden_states, router_weight, w1, w2):
    b, s, h = hidden_states.shape
    t = b * s
    dff = w1.shape[2]
    na = t * TOPK
    nblk = na // BT + E
    nrow = nblk * BT
    flat = hidden_states.reshape(t, h)

    idx2, prob2 = pl.pallas_call(
        _router_kernel,
        out_shape=(jax.ShapeDtypeStruct((2, t), jnp.int32),
                   jax.ShapeDtypeStruct((2, t), jnp.float32)),
    )(flat, router_weight)
    a_flat = idx2.T.reshape(na)
    pr_flat = prob2.T.reshape(na)

    mesh = plsc.VectorSubcoreMesh(core_axis_name="c", subcore_axis_name="s",
                                  num_cores=2)
    nw = 32

    tok_sorted, prob_sorted, pos, off_incl = pl.kernel(
        _dispatch_body,
        out_type=(jax.ShapeDtypeStruct((nrow,), jnp.int32),
                  jax.ShapeDtypeStruct((nrow,), jnp.float32),
                  jax.ShapeDtypeStruct((na,), jnp.int32),
                  jax.ShapeDtypeStruct((16,), jnp.int32)),
        mesh=mesh,
        compiler_params=pltpu.CompilerParams(needs_layout_passes=False),
        scratch_types=[
            pltpu.VMEM((na,), jnp.int32),
            pltpu.VMEM((na,), jnp.float32),
            pltpu.VMEM((nrow,), jnp.int32),
            pltpu.VMEM((nrow,), jnp.float32),
            pltpu.VMEM((na,), jnp.int32),
            pltpu.VMEM((16,), jnp.int32),
            pltpu.VMEM((16,), jnp.int32),
        ],
    )(a_flat, pr_flat)

    xs = pl.kernel(
        _gather_body,
        out_type=jax.ShapeDtypeStruct((nrow, h), jnp.float32),
        mesh=mesh,
        compiler_params=pltpu.CompilerParams(needs_layout_passes=False),
        scratch_types=[
            pltpu.VMEM((48,), jnp.int32),
            pltpu.VMEM((48,), jnp.int32),
            pltpu.VMEM((48,), jnp.int32),
            pltpu.VMEM((48,), jnp.int32),
            pltpu.VMEM((48, h), jnp.float32),
            pltpu.VMEM((48, h), jnp.float32),
            pltpu.SemaphoreType.DMA,
            pltpu.SemaphoreType.DMA,
            pltpu.SemaphoreType.DMA,
        ],
    )(flat, tok_sorted)

    blk = jnp.arange(nblk, dtype=jnp.int32) * BT
    eid = jnp.sum((blk[:, None] >= off_incl[None, :E]).astype(jnp.int32),
                  axis=1)
    eid = jnp.minimum(eid, E - 1)

    ys = pl.pallas_call(
        _ffn_kernel,
        grid_spec=pltpu.PrefetchScalarGridSpec(
            num_scalar_prefetch=1,
            grid=(nblk,),
            in_specs=[
                pl.BlockSpec((BT, h), lambda i, eid_ref: (i, 0)),
                pl.BlockSpec((1, h, dff), lambda i, eid_ref: (eid_ref[i], 0, 0)),
                pl.BlockSpec((1, dff, h), lambda i, eid_ref: (eid_ref[i], 0, 0)),
                pl.BlockSpec((BT, 1), lambda i, eid_ref: (i, 0)),
            ],
            out_specs=pl.BlockSpec((BT, h), lambda i, eid_ref: (i, 0)),
            scratch_shapes=[
                pltpu.VMEM((h, dff), jnp.bfloat16),
                pltpu.VMEM((dff, h), jnp.bfloat16),
            ],
        ),
        out_shape=jax.ShapeDtypeStruct((nrow, h), jnp.float32),
    )(eid, xs, w1, w2, prob_sorted.reshape(nrow, 1))

    pos2d = pos.reshape(t, TOPK)
    pos_a = pos2d[:, 0]
    pos_b = pos2d[:, 1]

    out = pl.kernel(
        _combine_body,
        out_type=jax.ShapeDtypeStruct((t, h), jnp.float32),
        mesh=mesh,
        compiler_params=pltpu.CompilerParams(needs_layout_passes=False),
        scratch_types=[
            pltpu.VMEM((32,), jnp.int32),
            pltpu.VMEM((32,), jnp.int32),
            pltpu.VMEM((32,), jnp.int32),
            pltpu.VMEM((32,), jnp.int32),
            pltpu.VMEM((32, h), jnp.float32),
            pltpu.VMEM((32, h), jnp.float32),
            pltpu.SemaphoreType.DMA,
            pltpu.SemaphoreType.DMA,
            pltpu.SemaphoreType.DMA,
        ],
    )(ys, pos_a, pos_b)
    return out.reshape(b, s, h)


# speedup vs baseline: 2.3755x; 2.2546x over previous
import jax
import jax.numpy as jnp
from jax import lax
from jax.experimental import pallas as pl

E = 8
_GELU_C = 0.7978845608028654


def _gelu_tanh(x):
    return 0.5 * x * (1.0 + jnp.tanh(_GELU_C * (x + 0.044715 * x * x * x)))


def _router_kernel(flat_ref, rw_ref, comb_ref, xb_ref):
    logits = lax.dot_general(
        rw_ref[...], flat_ref[...], (((1,), (1,)), ((), ())),
        preferred_element_type=jnp.float32)
    e, t = logits.shape
    ids = lax.broadcasted_iota(jnp.int32, (e, t), 0)
    m1 = jnp.max(logits, axis=0, keepdims=True)
    a1 = jnp.min(jnp.where(logits == m1, ids, e), axis=0, keepdims=True)
    l2 = jnp.where(ids == a1, -jnp.inf, logits)
    m2 = jnp.max(l2, axis=0, keepdims=True)
    a2 = jnp.min(jnp.where(l2 == m2, ids, e), axis=0, keepdims=True)
    e2 = jnp.exp(m2 - m1)
    p1 = 1.0 / (1.0 + e2)
    p2 = e2 * p1
    comb_ref[...] = jnp.where(ids == a1, p1, 0.0) + jnp.where(ids == a2, p2, 0.0)
    xb_ref[...] = flat_ref[...].astype(jnp.bfloat16)


def _moe_dense_kernel(comb_ref, xb_ref, w1_ref, w2_ref, out_ref):
    e = pl.program_id(0)
    x = xb_ref[...]
    dff = w1_ref.shape[2]
    hf = dff // 2

    def half(lo):
        w1 = w1_ref[0, :, pl.ds(lo, hf)].astype(jnp.bfloat16)
        h = jnp.dot(x, w1, preferred_element_type=jnp.float32)
        h = _gelu_tanh(h).astype(jnp.bfloat16)
        w2 = w2_ref[0, pl.ds(lo, hf), :].astype(jnp.bfloat16)
        return jnp.dot(h, w2, preferred_element_type=jnp.float32)

    y = half(0) + half(hf)
    contrib = y * comb_ref[0]

    @pl.when(e == 0)
    def _():
        out_ref[...] = contrib

    @pl.when(e != 0)
    def _():
        out_ref[...] += contrib


def kernel(hidden_states, router_weight, w1, w2):
    b, s, h = hidden_states.shape
    t = b * s
    dff = w1.shape[2]
    flat = hidden_states.reshape(t, h)

    comb, xb = pl.pallas_call(
        _router_kernel,
        out_shape=(jax.ShapeDtypeStruct((E, t), jnp.float32),
                   jax.ShapeDtypeStruct((t, h), jnp.bfloat16)),
    )(flat, router_weight)
    comb = comb.reshape(E, t, 1)

    out = pl.pallas_call(
        _moe_dense_kernel,
        grid=(E,),
        in_specs=[
            pl.BlockSpec((1, t, 1), lambda e: (e, 0, 0)),
            pl.BlockSpec((t, h), lambda e: (0, 0)),
            pl.BlockSpec((1, h, dff), lambda e: (e, 0, 0)),
            pl.BlockSpec((1, dff, h), lambda e: (e, 0, 0)),
        ],
        out_specs=pl.BlockSpec((t, h), lambda e: (0, 0)),
        out_shape=jax.ShapeDtypeStruct((t, h), jnp.float32),
    )(comb, xb, w1, w2)
    return out.reshape(b, s, h)
